# baseline (speedup 1.0000x reference)
import functools

import jax
import jax.numpy as jnp
from jax import lax
from jax.experimental import pallas as pl
from jax.experimental.pallas import tpu as pltpu
from jax.experimental.pallas import tpu_sc as plsc

NUM_TOKENS = 1000000
BATCH = 16384
SEQ_LEN = 50
DIM = 64
SPLIT = 500224

_info = plsc.get_sparse_core_info()
_NC, _NS, _L = _info.num_cores, _info.num_subcores, _info.num_lanes
_NW = _NC * _NS

_SAMPLES_PER_ROW = 2
_IDX_PER_ROW = _SAMPLES_PER_ROW * SEQ_LEN
_ROWS_PER_BLOCK = 4
_SAMPLES_PER_BLOCK = _ROWS_PER_BLOCK * _SAMPLES_PER_ROW
_GROWS_PER_BLOCK = _ROWS_PER_BLOCK * _IDX_PER_ROW
_SAMPLES_PER_WORKER = BATCH // _NW
_BLOCKS_PER_WORKER = _SAMPLES_PER_WORKER // _SAMPLES_PER_BLOCK
_X_ROWS_PER_WORKER = _SAMPLES_PER_WORKER // _SAMPLES_PER_ROW

_TC_COLS = 512


def _transpose_body(a_ref, b_ref, out_ref):
    out_ref[:, 0:DIM] = jnp.transpose(a_ref[...], (1, 0))
    out_ref[:, DIM:2 * DIM] = jnp.transpose(b_ref[...], (1, 0))


def _repack_table(tT):
    nblk = SPLIT // _TC_COLS
    return pl.pallas_call(
        _transpose_body,
        grid=(nblk,),
        in_specs=[
            pl.BlockSpec((DIM, _TC_COLS), lambda i: (0, i)),
            pl.BlockSpec((DIM, _TC_COLS), lambda i: (0, i + nblk)),
        ],
        out_specs=pl.BlockSpec((_TC_COLS, 2 * DIM), lambda i: (i, 0)),
        out_shape=jax.ShapeDtypeStruct((SPLIT, 2 * DIM), jnp.float32),
    )(tT, tT)


def _bow_body(xq_hbm, xh_hbm, t3_hbm, bias_hbm, out_hbm,
              idx_v, xh_v, rows_v, out_v, bias_v,
              sem0, sem1, osem0, osem1):
    wid = lax.axis_index("s") * _NC + lax.axis_index("c")
    sems = (sem0, sem1)
    osems = (osem0, osem1)

    pltpu.sync_copy(bias_hbm, bias_v)

    def fire(slot, b):
        rb = wid * _X_ROWS_PER_WORKER + b * _ROWS_PER_BLOCK
        pltpu.sync_copy(xq_hbm.at[pl.ds(rb, _ROWS_PER_BLOCK), :],
                        idx_v.at[slot])
        sb = wid * _SAMPLES_PER_WORKER + b * _SAMPLES_PER_BLOCK
        pltpu.sync_copy(xh_hbm.at[pl.ds(sb, _SAMPLES_PER_BLOCK), :],
                        xh_v.at[slot])
        for j in range(_ROWS_PER_BLOCK):
            pltpu.async_copy(
                t3_hbm.at[idx_v.at[slot, j]],
                rows_v.at[slot, pl.ds(j * _IDX_PER_ROW, _IDX_PER_ROW), :],
                sems[slot])

    def drain(slot):
        for j in range(_ROWS_PER_BLOCK):
            pltpu.make_async_copy(
                t3_hbm.at[idx_v.at[slot, j]],
                rows_v.at[slot, pl.ds(j * _IDX_PER_ROW, _IDX_PER_ROW), :],
                sems[slot]).wait()

    def compute(slot, b):
        sb = wid * _SAMPLES_PER_WORKER + b * _SAMPLES_PER_BLOCK

        def sample_body(s, _):
            rb = s * SEQ_LEN
            hs = [xh_v[slot, s, pl.ds(k * _L, _L)] for k in range(4)]
            accs = [bias_v[pl.ds(g * _L, _L)] for g in range(4)]
            for r in range(SEQ_LEN):
                off = hs[r // _L][r % _L]
                for g in range(4):
                    accs[g] = accs[g] + rows_v[slot, rb + r,
                                               pl.ds(off + g * _L, _L)]
            for g in range(4):
                out_v[slot, s, pl.ds(g * _L, _L)] = accs[g]
            return 0

        lax.fori_loop(0, _SAMPLES_PER_BLOCK, sample_body, 0)
        pltpu.async_copy(out_v.at[slot],
                         out_hbm.at[pl.ds(sb, _SAMPLES_PER_BLOCK), :],
                         osems[slot])

    def drain_out(slot, b):
        sb = wid * _SAMPLES_PER_WORKER + b * _SAMPLES_PER_BLOCK
        pltpu.make_async_copy(
            out_v.at[slot],
            out_hbm.at[pl.ds(sb, _SAMPLES_PER_BLOCK), :],
            osems[slot]).wait()

    fire(0, 0)

    def pair_body(i, _):
        for phase in range(2):
            b = 2 * i + phase
            cur, nxt = phase, 1 - phase

            @pl.when(b + 1 < _BLOCKS_PER_WORKER)
            def _():
                fire(nxt, b + 1)

            drain(cur)

            @pl.when(b >= 2)
            def _():
                drain_out(cur, b - 2)

            compute(cur, b)
        return 0

    lax.fori_loop(0, _BLOCKS_PER_WORKER // 2, pair_body, 0)
    drain_out(0, _BLOCKS_PER_WORKER - 2)
    drain_out(1, _BLOCKS_PER_WORKER - 1)


@jax.jit
def _bow_call(x, table, bias):
    tT = jnp.transpose(table)
    t3 = _repack_table(tT)

    x = x.astype(jnp.int32)
    xq = (x % SPLIT).reshape(BATCH // _SAMPLES_PER_ROW, _IDX_PER_ROW)
    xh = jnp.pad((x // SPLIT) << 6, ((0, 0), (0, DIM - SEQ_LEN)))

    mesh = plsc.VectorSubcoreMesh(core_axis_name="c", subcore_axis_name="s")
    f = functools.partial(
        pl.kernel,
        mesh=mesh,
        out_type=jax.ShapeDtypeStruct((BATCH, DIM), jnp.float32),
        scratch_types=[
            pltpu.VMEM((2, _ROWS_PER_BLOCK, _IDX_PER_ROW), jnp.int32),
            pltpu.VMEM((2, _SAMPLES_PER_BLOCK, DIM), jnp.int32),
            pltpu.VMEM((2, _GROWS_PER_BLOCK, 2 * DIM), jnp.float32),
            pltpu.VMEM((2, _SAMPLES_PER_BLOCK, DIM), jnp.float32),
            pltpu.VMEM((DIM,), jnp.float32),
            pltpu.SemaphoreType.DMA,
            pltpu.SemaphoreType.DMA,
            pltpu.SemaphoreType.DMA,
            pltpu.SemaphoreType.DMA,
        ],
        compiler_params=pltpu.CompilerParams(use_tc_tiling_on_sc=True),
    )(_bow_body)
    return f(xq, xh, t3, bias)


def kernel(x, table, bias):
    logits = _bow_call(x, table, bias)
    return (logits[:, :16], logits[:, 16:32], logits[:, 32:])


# speedup vs baseline: 1.3164x; 1.3164x over previous
import functools

import jax
import jax.numpy as jnp
from jax import lax
from jax.experimental import pallas as pl
from jax.experimental.pallas import tpu as pltpu
from jax.experimental.pallas import tpu_sc as plsc

NUM_TOKENS = 1000000
BATCH = 16384
SEQ_LEN = 50
DIM = 64
SPLIT = 500736

_info = plsc.get_sparse_core_info()
_NC, _NS, _L = _info.num_cores, _info.num_subcores, _info.num_lanes
_NW = _NC * _NS

_SAMPLES_PER_ROW = 2
_IDX_PER_ROW = _SAMPLES_PER_ROW * SEQ_LEN
_ROWS_PER_BLOCK = 4
_SAMPLES_PER_BLOCK = _ROWS_PER_BLOCK * _SAMPLES_PER_ROW
_GROWS_PER_BLOCK = _ROWS_PER_BLOCK * _IDX_PER_ROW
_SAMPLES_PER_WORKER = BATCH // _NW
_BLOCKS_PER_WORKER = _SAMPLES_PER_WORKER // _SAMPLES_PER_BLOCK
_X_ROWS_PER_WORKER = _SAMPLES_PER_WORKER // _SAMPLES_PER_ROW

_TC_COLS = 1024
_TC_LAST_BLK = (NUM_TOKENS + _TC_COLS - 1) // _TC_COLS - 1


def _transpose_body(a_ref, b_ref, out_ref):
    out_ref[:, 0:DIM] = jnp.transpose(a_ref[...], (1, 0))
    out_ref[:, DIM:2 * DIM] = jnp.transpose(b_ref[...], (1, 0))


def _repack_table(tT):
    nblk = SPLIT // _TC_COLS
    return pl.pallas_call(
        _transpose_body,
        grid=(nblk,),
        in_specs=[
            pl.BlockSpec((DIM, _TC_COLS), lambda i: (0, i)),
            pl.BlockSpec((DIM, _TC_COLS),
                         lambda i: (0, jnp.minimum(i + nblk, _TC_LAST_BLK))),
        ],
        out_specs=pl.BlockSpec((_TC_COLS, 2 * DIM), lambda i: (i, 0)),
        out_shape=jax.ShapeDtypeStruct((SPLIT, 2 * DIM), jnp.float32),
    )(tT, tT)


def _bow_body(xq_hbm, xh_hbm, t3_hbm, bias_hbm, out_hbm,
              idx_v, xh_v, rows_v, out_v, bias_v,
              sem0, sem1, osem0, osem1):
    wid = lax.axis_index("s") * _NC + lax.axis_index("c")
    sems = (sem0, sem1)
    osems = (osem0, osem1)

    pltpu.sync_copy(bias_hbm, bias_v)

    def fire(slot, b):
        rb = wid * _X_ROWS_PER_WORKER + b * _ROWS_PER_BLOCK
        pltpu.sync_copy(xq_hbm.at[pl.ds(rb, _ROWS_PER_BLOCK), :],
                        idx_v.at[slot])
        sb = wid * _SAMPLES_PER_WORKER + b * _SAMPLES_PER_BLOCK
        pltpu.sync_copy(xh_hbm.at[pl.ds(sb, _SAMPLES_PER_BLOCK), :],
                        xh_v.at[slot])
        for j in range(_ROWS_PER_BLOCK):
            pltpu.async_copy(
                t3_hbm.at[idx_v.at[slot, j]],
                rows_v.at[slot, pl.ds(j * _IDX_PER_ROW, _IDX_PER_ROW), :],
                sems[slot])

    def drain(slot):
        for j in range(_ROWS_PER_BLOCK):
            pltpu.make_async_copy(
                t3_hbm.at[idx_v.at[slot, j]],
                rows_v.at[slot, pl.ds(j * _IDX_PER_ROW, _IDX_PER_ROW), :],
                sems[slot]).wait()

    def compute(slot, b):
        sb = wid * _SAMPLES_PER_WORKER + b * _SAMPLES_PER_BLOCK

        def sample_body(s, _):
            rb = s * SEQ_LEN
            hs = [xh_v[slot, s, pl.ds(k * _L, _L)] for k in range(4)]
            accs = [bias_v[pl.ds(g * _L, _L)] for g in range(4)]
            for r in range(SEQ_LEN):
                off = hs[r // _L][r % _L]
                for g in range(4):
                    accs[g] = accs[g] + rows_v[slot, rb + r,
                                               pl.ds(off + g * _L, _L)]
            for g in range(4):
                out_v[slot, s, pl.ds(g * _L, _L)] = accs[g]
            return 0

        lax.fori_loop(0, _SAMPLES_PER_BLOCK, sample_body, 0)
        pltpu.async_copy(out_v.at[slot],
                         out_hbm.at[pl.ds(sb, _SAMPLES_PER_BLOCK), :],
                         osems[slot])

    def drain_out(slot, b):
        sb = wid * _SAMPLES_PER_WORKER + b * _SAMPLES_PER_BLOCK
        pltpu.make_async_copy(
            out_v.at[slot],
            out_hbm.at[pl.ds(sb, _SAMPLES_PER_BLOCK), :],
            osems[slot]).wait()

    fire(0, 0)

    def pair_body(i, _):
        for phase in range(2):
            b = 2 * i + phase
            cur, nxt = phase, 1 - phase

            @pl.when(b + 1 < _BLOCKS_PER_WORKER)
            def _():
                fire(nxt, b + 1)

            drain(cur)

            @pl.when(b >= 2)
            def _():
                drain_out(cur, b - 2)

            compute(cur, b)
        return 0

    lax.fori_loop(0, _BLOCKS_PER_WORKER // 2, pair_body, 0)
    drain_out(0, _BLOCKS_PER_WORKER - 2)
    drain_out(1, _BLOCKS_PER_WORKER - 1)


@jax.jit
def _bow_call(x, table, bias):
    tT = jnp.transpose(table)
    t3 = _repack_table(tT)

    x = x.astype(jnp.int32)
    xq = (x % SPLIT).reshape(BATCH // _SAMPLES_PER_ROW, _IDX_PER_ROW)
    xh = jnp.pad((x // SPLIT) << 6, ((0, 0), (0, DIM - SEQ_LEN)))

    mesh = plsc.VectorSubcoreMesh(core_axis_name="c", subcore_axis_name="s")
    f = functools.partial(
        pl.kernel,
        mesh=mesh,
        out_type=jax.ShapeDtypeStruct((BATCH, DIM), jnp.float32),
        scratch_types=[
            pltpu.VMEM((2, _ROWS_PER_BLOCK, _IDX_PER_ROW), jnp.int32),
            pltpu.VMEM((2, _SAMPLES_PER_BLOCK, DIM), jnp.int32),
            pltpu.VMEM((2, _GROWS_PER_BLOCK, 2 * DIM), jnp.float32),
            pltpu.VMEM((2, _SAMPLES_PER_BLOCK, DIM), jnp.float32),
            pltpu.VMEM((DIM,), jnp.float32),
            pltpu.SemaphoreType.DMA,
            pltpu.SemaphoreType.DMA,
            pltpu.SemaphoreType.DMA,
            pltpu.SemaphoreType.DMA,
        ],
        compiler_params=pltpu.CompilerParams(use_tc_tiling_on_sc=True),
    )(_bow_body)
    return f(xq, xh, t3, bias)


def kernel(x, table, bias):
    logits = _bow_call(x, table, bias)
    return (logits[:, :16], logits[:, 16:32], logits[:, 32:])


# speedup vs baseline: 1.5673x; 1.1906x over previous
import functools

import jax
import jax.numpy as jnp
from jax import lax
from jax.experimental import pallas as pl
from jax.experimental.pallas import tpu as pltpu
from jax.experimental.pallas import tpu_sc as plsc

NUM_TOKENS = 1000000
BATCH = 16384
SEQ_LEN = 50
DIM = 64
SPLIT = 501760

_info = plsc.get_sparse_core_info()
_NC, _NS, _L = _info.num_cores, _info.num_subcores, _info.num_lanes
_NW = _NC * _NS

_SAMPLES_PER_ROW = 2
_IDX_PER_ROW = _SAMPLES_PER_ROW * SEQ_LEN
_ROWS_PER_BLOCK = 4
_SAMPLES_PER_BLOCK = _ROWS_PER_BLOCK * _SAMPLES_PER_ROW
_GROWS_PER_BLOCK = _ROWS_PER_BLOCK * _IDX_PER_ROW
_SAMPLES_PER_WORKER = BATCH // _NW
_BLOCKS_PER_WORKER = _SAMPLES_PER_WORKER // _SAMPLES_PER_BLOCK
_X_ROWS_PER_WORKER = _SAMPLES_PER_WORKER // _SAMPLES_PER_ROW

_TC_COLS = 2048
_TC_LAST_BLK = (NUM_TOKENS + _TC_COLS - 1) // _TC_COLS - 1


def _transpose_body(a_ref, b_ref, out_ref):
    out_ref[:, 0:DIM] = jnp.transpose(a_ref[...], (1, 0))
    out_ref[:, DIM:2 * DIM] = jnp.transpose(b_ref[...], (1, 0))


def _repack_table(tT):
    nblk = SPLIT // _TC_COLS
    return pl.pallas_call(
        _transpose_body,
        grid=(nblk,),
        in_specs=[
            pl.BlockSpec((DIM, _TC_COLS), lambda i: (0, i)),
            pl.BlockSpec((DIM, _TC_COLS),
                         lambda i: (0, jnp.minimum(i + nblk, _TC_LAST_BLK))),
        ],
        out_specs=pl.BlockSpec((_TC_COLS, 2 * DIM), lambda i: (i, 0)),
        out_shape=jax.ShapeDtypeStruct((SPLIT, 2 * DIM), jnp.float32),
    )(tT, tT)


def _bow_body(xq_hbm, xh_hbm, t3_hbm, bias_hbm, out_hbm,
              idx_v, xh_v, rows_v, out_v, bias_v,
              sem0, sem1, osem0, osem1):
    wid = lax.axis_index("s") * _NC + lax.axis_index("c")
    sems = (sem0, sem1)
    osems = (osem0, osem1)

    pltpu.sync_copy(bias_hbm, bias_v)

    def fire(slot, b):
        rb = wid * _X_ROWS_PER_WORKER + b * _ROWS_PER_BLOCK
        pltpu.sync_copy(xq_hbm.at[pl.ds(rb, _ROWS_PER_BLOCK), :],
                        idx_v.at[slot])
        sb = wid * _SAMPLES_PER_WORKER + b * _SAMPLES_PER_BLOCK
        pltpu.sync_copy(xh_hbm.at[pl.ds(sb, _SAMPLES_PER_BLOCK), :],
                        xh_v.at[slot])
        for j in range(_ROWS_PER_BLOCK):
            pltpu.async_copy(
                t3_hbm.at[idx_v.at[slot, j]],
                rows_v.at[slot, pl.ds(j * _IDX_PER_ROW, _IDX_PER_ROW), :],
                sems[slot])

    def drain(slot):
        for j in range(_ROWS_PER_BLOCK):
            pltpu.make_async_copy(
                t3_hbm.at[idx_v.at[slot, j]],
                rows_v.at[slot, pl.ds(j * _IDX_PER_ROW, _IDX_PER_ROW), :],
                sems[slot]).wait()

    def compute(slot, b):
        sb = wid * _SAMPLES_PER_WORKER + b * _SAMPLES_PER_BLOCK

        def sample_body(s, _):
            rb = s * SEQ_LEN
            hs = [xh_v[slot, s, pl.ds(k * _L, _L)] for k in range(4)]
            accs = [bias_v[pl.ds(g * _L, _L)] for g in range(4)]
            for r in range(SEQ_LEN):
                off = hs[r // _L][r % _L]
                for g in range(4):
                    accs[g] = accs[g] + rows_v[slot, rb + r,
                                               pl.ds(off + g * _L, _L)]
            for g in range(4):
                out_v[slot, s, pl.ds(g * _L, _L)] = accs[g]
            return 0

        lax.fori_loop(0, _SAMPLES_PER_BLOCK, sample_body, 0)
        pltpu.async_copy(out_v.at[slot],
                         out_hbm.at[pl.ds(sb, _SAMPLES_PER_BLOCK), :],
                         osems[slot])

    def drain_out(slot, b):
        sb = wid * _SAMPLES_PER_WORKER + b * _SAMPLES_PER_BLOCK
        pltpu.make_async_copy(
            out_v.at[slot],
            out_hbm.at[pl.ds(sb, _SAMPLES_PER_BLOCK), :],
            osems[slot]).wait()

    fire(0, 0)

    def pair_body(i, _):
        for phase in range(2):
            b = 2 * i + phase
            cur, nxt = phase, 1 - phase

            @pl.when(b + 1 < _BLOCKS_PER_WORKER)
            def _():
                fire(nxt, b + 1)

            drain(cur)

            @pl.when(b >= 2)
            def _():
                drain_out(cur, b - 2)

            compute(cur, b)
        return 0

    lax.fori_loop(0, _BLOCKS_PER_WORKER // 2, pair_body, 0)
    drain_out(0, _BLOCKS_PER_WORKER - 2)
    drain_out(1, _BLOCKS_PER_WORKER - 1)


@jax.jit
def _bow_call(x, table, bias):
    tT = jnp.transpose(table)
    t3 = _repack_table(tT)

    x = x.astype(jnp.int32)
    xq = (x % SPLIT).reshape(BATCH // _SAMPLES_PER_ROW, _IDX_PER_ROW)
    xh = jnp.pad((x // SPLIT) << 6, ((0, 0), (0, DIM - SEQ_LEN)))

    mesh = plsc.VectorSubcoreMesh(core_axis_name="c", subcore_axis_name="s")
    f = functools.partial(
        pl.kernel,
        mesh=mesh,
        out_type=jax.ShapeDtypeStruct((BATCH, DIM), jnp.float32),
        scratch_types=[
            pltpu.VMEM((2, _ROWS_PER_BLOCK, _IDX_PER_ROW), jnp.int32),
            pltpu.VMEM((2, _SAMPLES_PER_BLOCK, DIM), jnp.int32),
            pltpu.VMEM((2, _GROWS_PER_BLOCK, 2 * DIM), jnp.float32),
            pltpu.VMEM((2, _SAMPLES_PER_BLOCK, DIM), jnp.float32),
            pltpu.VMEM((DIM,), jnp.float32),
            pltpu.SemaphoreType.DMA,
            pltpu.SemaphoreType.DMA,
            pltpu.SemaphoreType.DMA,
            pltpu.SemaphoreType.DMA,
        ],
        compiler_params=pltpu.CompilerParams(use_tc_tiling_on_sc=True),
    )(_bow_body)
    return f(xq, xh, t3, bias)


def kernel(x, table, bias):
    logits = _bow_call(x, table, bias)
    return (logits[:, :16], logits[:, 16:32], logits[:, 32:])


# speedup vs baseline: 1.7578x; 1.1215x over previous
import functools

import jax
import jax.numpy as jnp
from jax import lax
from jax.experimental import pallas as pl
from jax.experimental.pallas import tpu as pltpu
from jax.experimental.pallas import tpu_sc as plsc

NUM_TOKENS = 1000000
BATCH = 16384
SEQ_LEN = 50
DIM = 64
SPLIT = 503808

_info = plsc.get_sparse_core_info()
_NC, _NS, _L = _info.num_cores, _info.num_subcores, _info.num_lanes
_NW = _NC * _NS

_SAMPLES_PER_ROW = 2
_IDX_PER_ROW = _SAMPLES_PER_ROW * SEQ_LEN
_ROWS_PER_BLOCK = 4
_SAMPLES_PER_BLOCK = _ROWS_PER_BLOCK * _SAMPLES_PER_ROW
_GROWS_PER_BLOCK = _ROWS_PER_BLOCK * _IDX_PER_ROW
_SAMPLES_PER_WORKER = BATCH // _NW
_BLOCKS_PER_WORKER = _SAMPLES_PER_WORKER // _SAMPLES_PER_BLOCK
_X_ROWS_PER_WORKER = _SAMPLES_PER_WORKER // _SAMPLES_PER_ROW

_TC_COLS = 4096
_TC_LAST_BLK = (NUM_TOKENS + _TC_COLS - 1) // _TC_COLS - 1


def _transpose_body(a_ref, b_ref, out_ref):
    out_ref[:, 0:DIM] = jnp.transpose(a_ref[...], (1, 0))
    out_ref[:, DIM:2 * DIM] = jnp.transpose(b_ref[...], (1, 0))


def _repack_table(tT):
    nblk = SPLIT // _TC_COLS
    return pl.pallas_call(
        _transpose_body,
        grid=(nblk,),
        in_specs=[
            pl.BlockSpec((DIM, _TC_COLS), lambda i: (0, i)),
            pl.BlockSpec((DIM, _TC_COLS),
                         lambda i: (0, jnp.minimum(i + nblk, _TC_LAST_BLK))),
        ],
        out_specs=pl.BlockSpec((_TC_COLS, 2 * DIM), lambda i: (i, 0)),
        out_shape=jax.ShapeDtypeStruct((SPLIT, 2 * DIM), jnp.float32),
    )(tT, tT)


def _bow_body(xq_hbm, xh_hbm, t3_hbm, bias_hbm, out_hbm,
              idx_v, xh_v, rows_v, out_v, bias_v,
              sem0, sem1, osem0, osem1):
    wid = lax.axis_index("s") * _NC + lax.axis_index("c")
    sems = (sem0, sem1)
    osems = (osem0, osem1)

    pltpu.sync_copy(bias_hbm, bias_v)

    def fire(slot, b):
        rb = wid * _X_ROWS_PER_WORKER + b * _ROWS_PER_BLOCK
        pltpu.sync_copy(xq_hbm.at[pl.ds(rb, _ROWS_PER_BLOCK), :],
                        idx_v.at[slot])
        sb = wid * _SAMPLES_PER_WORKER + b * _SAMPLES_PER_BLOCK
        pltpu.sync_copy(xh_hbm.at[pl.ds(sb, _SAMPLES_PER_BLOCK), :],
                        xh_v.at[slot])
        for j in range(_ROWS_PER_BLOCK):
            pltpu.async_copy(
                t3_hbm.at[idx_v.at[slot, j]],
                rows_v.at[slot, pl.ds(j * _IDX_PER_ROW, _IDX_PER_ROW), :],
                sems[slot])

    def drain(slot):
        for j in range(_ROWS_PER_BLOCK):
            pltpu.make_async_copy(
                t3_hbm.at[idx_v.at[slot, j]],
                rows_v.at[slot, pl.ds(j * _IDX_PER_ROW, _IDX_PER_ROW), :],
                sems[slot]).wait()

    def compute(slot, b):
        sb = wid * _SAMPLES_PER_WORKER + b * _SAMPLES_PER_BLOCK

        def sample_body(s, _):
            rb = s * SEQ_LEN
            hs = [xh_v[slot, s, pl.ds(k * _L, _L)] for k in range(4)]
            accs = [bias_v[pl.ds(g * _L, _L)] for g in range(4)]
            for r in range(SEQ_LEN):
                off = hs[r // _L][r % _L]
                for g in range(4):
                    accs[g] = accs[g] + rows_v[slot, rb + r,
                                               pl.ds(off + g * _L, _L)]
            for g in range(4):
                out_v[slot, s, pl.ds(g * _L, _L)] = accs[g]
            return 0

        lax.fori_loop(0, _SAMPLES_PER_BLOCK, sample_body, 0)
        pltpu.async_copy(out_v.at[slot],
                         out_hbm.at[pl.ds(sb, _SAMPLES_PER_BLOCK), :],
                         osems[slot])

    def drain_out(slot, b):
        sb = wid * _SAMPLES_PER_WORKER + b * _SAMPLES_PER_BLOCK
        pltpu.make_async_copy(
            out_v.at[slot],
            out_hbm.at[pl.ds(sb, _SAMPLES_PER_BLOCK), :],
            osems[slot]).wait()

    fire(0, 0)

    def pair_body(i, _):
        for phase in range(2):
            b = 2 * i + phase
            cur, nxt = phase, 1 - phase

            @pl.when(b + 1 < _BLOCKS_PER_WORKER)
            def _():
                fire(nxt, b + 1)

            drain(cur)

            @pl.when(b >= 2)
            def _():
                drain_out(cur, b - 2)

            compute(cur, b)
        return 0

    lax.fori_loop(0, _BLOCKS_PER_WORKER // 2, pair_body, 0)
    drain_out(0, _BLOCKS_PER_WORKER - 2)
    drain_out(1, _BLOCKS_PER_WORKER - 1)


@jax.jit
def _bow_call(x, table, bias):
    tT = jnp.transpose(table)
    t3 = _repack_table(tT)

    x = x.astype(jnp.int32)
    xq = (x % SPLIT).reshape(BATCH // _SAMPLES_PER_ROW, _IDX_PER_ROW)
    xh = jnp.pad((x // SPLIT) << 6, ((0, 0), (0, DIM - SEQ_LEN)))

    mesh = plsc.VectorSubcoreMesh(core_axis_name="c", subcore_axis_name="s")
    f = functools.partial(
        pl.kernel,
        mesh=mesh,
        out_type=jax.ShapeDtypeStruct((BATCH, DIM), jnp.float32),
        scratch_types=[
            pltpu.VMEM((2, _ROWS_PER_BLOCK, _IDX_PER_ROW), jnp.int32),
            pltpu.VMEM((2, _SAMPLES_PER_BLOCK, DIM), jnp.int32),
            pltpu.VMEM((2, _GROWS_PER_BLOCK, 2 * DIM), jnp.float32),
            pltpu.VMEM((2, _SAMPLES_PER_BLOCK, DIM), jnp.float32),
            pltpu.VMEM((DIM,), jnp.float32),
            pltpu.SemaphoreType.DMA,
            pltpu.SemaphoreType.DMA,
            pltpu.SemaphoreType.DMA,
            pltpu.SemaphoreType.DMA,
        ],
        compiler_params=pltpu.CompilerParams(use_tc_tiling_on_sc=True),
    )(_bow_body)
    return f(xq, xh, t3, bias)


def kernel(x, table, bias):
    logits = _bow_call(x, table, bias)
    return (logits[:, :16], logits[:, 16:32], logits[:, 32:])


# speedup vs baseline: 1.8588x; 1.0575x over previous
import functools

import jax
import jax.numpy as jnp
from jax import lax
from jax.experimental import pallas as pl
from jax.experimental.pallas import tpu as pltpu
from jax.experimental.pallas import tpu_sc as plsc

NUM_TOKENS = 1000000
BATCH = 16384
SEQ_LEN = 50
DIM = 64
SPLIT = 507904

_info = plsc.get_sparse_core_info()
_NC, _NS, _L = _info.num_cores, _info.num_subcores, _info.num_lanes
_NW = _NC * _NS

_SAMPLES_PER_ROW = 2
_IDX_PER_ROW = _SAMPLES_PER_ROW * SEQ_LEN
_ROWS_PER_BLOCK = 4
_SAMPLES_PER_BLOCK = _ROWS_PER_BLOCK * _SAMPLES_PER_ROW
_GROWS_PER_BLOCK = _ROWS_PER_BLOCK * _IDX_PER_ROW
_SAMPLES_PER_WORKER = BATCH // _NW
_BLOCKS_PER_WORKER = _SAMPLES_PER_WORKER // _SAMPLES_PER_BLOCK
_X_ROWS_PER_WORKER = _SAMPLES_PER_WORKER // _SAMPLES_PER_ROW

_TC_COLS = 8192
_TC_LAST_BLK = (NUM_TOKENS + _TC_COLS - 1) // _TC_COLS - 1


def _transpose_body(a_ref, b_ref, out_ref):
    out_ref[:, 0:DIM] = jnp.transpose(a_ref[...], (1, 0))
    out_ref[:, DIM:2 * DIM] = jnp.transpose(b_ref[...], (1, 0))


def _repack_table(tT):
    nblk = SPLIT // _TC_COLS
    return pl.pallas_call(
        _transpose_body,
        grid=(nblk,),
        in_specs=[
            pl.BlockSpec((DIM, _TC_COLS), lambda i: (0, i)),
            pl.BlockSpec((DIM, _TC_COLS),
                         lambda i: (0, jnp.minimum(i + nblk, _TC_LAST_BLK))),
        ],
        out_specs=pl.BlockSpec((_TC_COLS, 2 * DIM), lambda i: (i, 0)),
        out_shape=jax.ShapeDtypeStruct((SPLIT, 2 * DIM), jnp.float32),
    )(tT, tT)


def _bow_body(xq_hbm, xh_hbm, t3_hbm, bias_hbm, out_hbm,
              idx_v, xh_v, rows_v, out_v, bias_v,
              sem0, sem1, osem0, osem1):
    wid = lax.axis_index("s") * _NC + lax.axis_index("c")
    sems = (sem0, sem1)
    osems = (osem0, osem1)

    pltpu.sync_copy(bias_hbm, bias_v)

    def fire(slot, b):
        rb = wid * _X_ROWS_PER_WORKER + b * _ROWS_PER_BLOCK
        pltpu.sync_copy(xq_hbm.at[pl.ds(rb, _ROWS_PER_BLOCK), :],
                        idx_v.at[slot])
        sb = wid * _SAMPLES_PER_WORKER + b * _SAMPLES_PER_BLOCK
        pltpu.sync_copy(xh_hbm.at[pl.ds(sb, _SAMPLES_PER_BLOCK), :],
                        xh_v.at[slot])
        for j in range(_ROWS_PER_BLOCK):
            pltpu.async_copy(
                t3_hbm.at[idx_v.at[slot, j]],
                rows_v.at[slot, pl.ds(j * _IDX_PER_ROW, _IDX_PER_ROW), :],
                sems[slot])

    def drain(slot):
        for j in range(_ROWS_PER_BLOCK):
            pltpu.make_async_copy(
                t3_hbm.at[idx_v.at[slot, j]],
                rows_v.at[slot, pl.ds(j * _IDX_PER_ROW, _IDX_PER_ROW), :],
                sems[slot]).wait()

    def compute(slot, b):
        sb = wid * _SAMPLES_PER_WORKER + b * _SAMPLES_PER_BLOCK

        def sample_body(s, _):
            rb = s * SEQ_LEN
            hs = [xh_v[slot, s, pl.ds(k * _L, _L)] for k in range(4)]
            accs = [bias_v[pl.ds(g * _L, _L)] for g in range(4)]
            for r in range(SEQ_LEN):
                off = hs[r // _L][r % _L]
                for g in range(4):
                    accs[g] = accs[g] + rows_v[slot, rb + r,
                                               pl.ds(off + g * _L, _L)]
            for g in range(4):
                out_v[slot, s, pl.ds(g * _L, _L)] = accs[g]
            return 0

        lax.fori_loop(0, _SAMPLES_PER_BLOCK, sample_body, 0)
        pltpu.async_copy(out_v.at[slot],
                         out_hbm.at[pl.ds(sb, _SAMPLES_PER_BLOCK), :],
                         osems[slot])

    def drain_out(slot, b):
        sb = wid * _SAMPLES_PER_WORKER + b * _SAMPLES_PER_BLOCK
        pltpu.make_async_copy(
            out_v.at[slot],
            out_hbm.at[pl.ds(sb, _SAMPLES_PER_BLOCK), :],
            osems[slot]).wait()

    fire(0, 0)

    def pair_body(i, _):
        for phase in range(2):
            b = 2 * i + phase
            cur, nxt = phase, 1 - phase

            @pl.when(b + 1 < _BLOCKS_PER_WORKER)
            def _():
                fire(nxt, b + 1)

            drain(cur)

            @pl.when(b >= 2)
            def _():
                drain_out(cur, b - 2)

            compute(cur, b)
        return 0

    lax.fori_loop(0, _BLOCKS_PER_WORKER // 2, pair_body, 0)
    drain_out(0, _BLOCKS_PER_WORKER - 2)
    drain_out(1, _BLOCKS_PER_WORKER - 1)


@jax.jit
def _bow_call(x, table, bias):
    tT = jnp.transpose(table)
    t3 = _repack_table(tT)

    x = x.astype(jnp.int32)
    xq = (x % SPLIT).reshape(BATCH // _SAMPLES_PER_ROW, _IDX_PER_ROW)
    xh = jnp.pad((x // SPLIT) << 6, ((0, 0), (0, DIM - SEQ_LEN)))

    mesh = plsc.VectorSubcoreMesh(core_axis_name="c", subcore_axis_name="s")
    f = functools.partial(
        pl.kernel,
        mesh=mesh,
        out_type=jax.ShapeDtypeStruct((BATCH, DIM), jnp.float32),
        scratch_types=[
            pltpu.VMEM((2, _ROWS_PER_BLOCK, _IDX_PER_ROW), jnp.int32),
            pltpu.VMEM((2, _SAMPLES_PER_BLOCK, DIM), jnp.int32),
            pltpu.VMEM((2, _GROWS_PER_BLOCK, 2 * DIM), jnp.float32),
            pltpu.VMEM((2, _SAMPLES_PER_BLOCK, DIM), jnp.float32),
            pltpu.VMEM((DIM,), jnp.float32),
            pltpu.SemaphoreType.DMA,
            pltpu.SemaphoreType.DMA,
            pltpu.SemaphoreType.DMA,
            pltpu.SemaphoreType.DMA,
        ],
        compiler_params=pltpu.CompilerParams(use_tc_tiling_on_sc=True),
    )(_bow_body)
    return f(xq, xh, t3, bias)


def kernel(x, table, bias):
    logits = _bow_call(x, table, bias)
    return (logits[:, :16], logits[:, 16:32], logits[:, 32:])
